# Initial kernel scaffold; baseline (speedup 1.0000x reference)
import jax
import jax.numpy as jnp
from jax.experimental import pallas as pl


def kernel(atomic_numbers, alpha_table):
    raise NotImplementedError("write your pallas kernel here")


# speedup vs baseline: 398.7843x; 398.7843x over previous
import functools

import jax
import jax.numpy as jnp
from jax import lax
from jax.experimental import pallas as pl
from jax.experimental.pallas import tpu as pltpu
from jax.experimental.pallas import tpu_sc as plsc

_NORM = 0.1481847 / 14.3996

_NC = 2
_NS = 16
_NW = _NC * _NS
_L = 16

_TBL_PAD = 128


def _make_lookup(n):
    assert n % (8 * _NW) == 0
    per_w = n // _NW
    mesh = plsc.VectorSubcoreMesh(
        core_axis_name="c", subcore_axis_name="s",
        num_cores=_NC, num_subcores=_NS,
    )

    @functools.partial(
        pl.kernel,
        out_type=jax.ShapeDtypeStruct((n,), jnp.float32),
        mesh=mesh,
        scratch_types=[
            pltpu.VMEM((_TBL_PAD,), jnp.float32),
            pltpu.VMEM((per_w,), jnp.int32),
            pltpu.VMEM((per_w,), jnp.float32),
        ],
        compiler_params=pltpu.CompilerParams(needs_layout_passes=False),
    )
    def lookup(an_hbm, tbl_hbm, out_hbm, tbl_v, idx_v, val_v):
        wid = lax.axis_index("s") * _NC + lax.axis_index("c")
        base = wid * per_w

        pltpu.sync_copy(tbl_hbm, tbl_v)
        for j in range(_TBL_PAD // _L):
            sl = pl.ds(j * _L, _L)
            tbl_v[sl] = tbl_v[sl] * _NORM

        pltpu.sync_copy(an_hbm.at[pl.ds(base, per_w)], idx_v)

        @plsc.parallel_loop(0, per_w, step=_L, unroll=8)
        def _(i):
            sl = pl.ds(i, _L)
            val_v[sl] = plsc.load_gather(tbl_v, [idx_v[sl]])

        pltpu.sync_copy(val_v, out_hbm.at[pl.ds(base, per_w)])

    return lookup


def kernel(atomic_numbers, alpha_table):
    tbl = jnp.zeros((_TBL_PAD,), jnp.float32).at[: alpha_table.shape[0]].set(
        alpha_table
    )
    return _make_lookup(atomic_numbers.shape[0])(atomic_numbers, tbl)
